# baseline (speedup 1.0000x reference)
import jax
import jax.numpy as jnp
from jax import lax
from jax.experimental import pallas as pl
from jax.experimental.pallas import tpu as pltpu
from jax.experimental.pallas import tpu_sc as plsc

N_MID = 1000000
N_CAT = 100000
EMBED_DIM = 32
BATCH = 16384
MAX_LEN = 200

NW = 32
BATCH_PER_W = BATCH // NW
LPAD = 256
ROW_PAD = 128


def _body(mid_idx_hbm, cat_idx_hbm, mid_table, cat_table, out_hbm,
          midx_v, cidx_v, mrows_v, crows_v, stage_v, sem):
    wid = lax.axis_index("c") * 16 + lax.axis_index("s")
    b0 = wid * BATCH_PER_W

    def chunk(t, _):
        b = b0 + t
        pltpu.sync_copy(mid_idx_hbm.at[pl.ds(b, 1)], midx_v)
        pltpu.sync_copy(cat_idx_hbm.at[pl.ds(b, 1)], cidx_v)
        copies = []
        for s0 in (0, 128):
            cm = pltpu.make_async_copy(
                mid_table.at[midx_v.at[0, pl.ds(s0, 128)]],
                mrows_v.at[pl.ds(s0, 128), :], sem)
            cc = pltpu.make_async_copy(
                cat_table.at[cidx_v.at[0, pl.ds(s0, 128)]],
                crows_v.at[pl.ds(s0, 128), :], sem)
            cm.start()
            cc.start()
            copies.append(cm)
            copies.append(cc)
        for c in copies:
            c.wait()

        def assemble(l, _):
            stage_v[0, l, pl.ds(0, 16)] = mrows_v[l, pl.ds(0, 16)]
            stage_v[0, l, pl.ds(16, 16)] = mrows_v[l, pl.ds(16, 16)]
            stage_v[0, l, pl.ds(32, 16)] = crows_v[l, pl.ds(0, 16)]
            stage_v[0, l, pl.ds(48, 16)] = crows_v[l, pl.ds(16, 16)]
            return ()

        lax.fori_loop(0, MAX_LEN, assemble, (), unroll=4)
        pltpu.sync_copy(stage_v, out_hbm.at[pl.ds(b, 1)])
        return ()

    lax.fori_loop(0, BATCH_PER_W, chunk, (), unroll=False)


@jax.jit
def _run(mid_idx, cat_idx, mid_table_p, cat_table_p):
    mesh = plsc.VectorSubcoreMesh(core_axis_name="c", subcore_axis_name="s")
    f = pl.kernel(
        _body,
        out_type=jax.ShapeDtypeStruct((BATCH, MAX_LEN, 2 * EMBED_DIM),
                                      jnp.float32),
        mesh=mesh,
        scratch_types=[
            pltpu.VMEM((1, LPAD), jnp.int32),
            pltpu.VMEM((1, LPAD), jnp.int32),
            pltpu.VMEM((LPAD, ROW_PAD), jnp.float32),
            pltpu.VMEM((LPAD, ROW_PAD), jnp.float32),
            pltpu.VMEM((1, MAX_LEN, 2 * EMBED_DIM), jnp.float32),
            pltpu.SemaphoreType.DMA,
        ],
    )
    return f(mid_idx, cat_idx, mid_table_p, cat_table_p)


def kernel(mid_his_input, cat_his_input, mid_table, cat_table):
    mid_idx = mid_his_input.astype(jnp.int32)
    cat_idx = cat_his_input.astype(jnp.int32)
    mid_idx = jnp.concatenate([mid_idx, mid_idx[:, :LPAD - MAX_LEN]], axis=1)
    cat_idx = jnp.concatenate([cat_idx, cat_idx[:, :LPAD - MAX_LEN]], axis=1)
    mid_p = jnp.pad(mid_table, ((0, 0), (0, ROW_PAD - EMBED_DIM)))
    cat_p = jnp.pad(cat_table, ((0, 0), (0, ROW_PAD - EMBED_DIM)))
    return _run(mid_idx, cat_idx, mid_p, cat_p)
